# baseline (speedup 1.0000x reference)
import functools

import jax
import jax.numpy as jnp
from jax import lax
from jax.experimental import pallas as pl
from jax.experimental.pallas import tpu as pltpu
from jax.experimental.pallas import tpu_sc as plsc

_VOCAB = 68
_D = 512
_S = 2048
_B = 64
_NW = 32
_S_PER_W = _S // _NW
_CHUNK = 16
_VPT = _D // 16


def _positional_encoding():
    pos = jnp.arange(0, _S, 1, dtype=jnp.float32).reshape(_S, 1)
    two_i = jnp.arange(0, _D, 2, dtype=jnp.float32)
    denominator = jnp.power(10000.0, two_i / _D)
    even = jnp.sin(pos / denominator)
    odd = jnp.cos(pos / denominator)
    return jnp.stack((even, odd), axis=2).reshape(_S, _D)


@functools.partial(
    pl.kernel,
    out_type=jax.ShapeDtypeStruct((_B * _S * _D,), jnp.float32),
    mesh=plsc.VectorSubcoreMesh(core_axis_name="c", subcore_axis_name="s"),
    scratch_types=[
        pltpu.VMEM((_VOCAB * _D,), jnp.float32),
        pltpu.VMEM((_S_PER_W * _D,), jnp.float32),
        pltpu.VMEM((_CHUNK * _D,), jnp.float32),
        pltpu.VMEM((_CHUNK,), jnp.int32),
    ],
)
def _emb_kernel(
    x_hbm, table_hbm, pe_hbm, out_hbm, table_v, pe_v, buf, ids_v
):
    wid = lax.axis_index("s") * 2 + lax.axis_index("c")
    s0 = wid * _S_PER_W

    pltpu.sync_copy(table_hbm, table_v)
    pltpu.sync_copy(pe_hbm.at[pl.ds(s0 * _D, _S_PER_W * _D)], pe_v)

    n_chunks = _S_PER_W // _CHUNK

    def chunk_body(i, carry):
        b = i // n_chunks
        c = i % n_chunks
        tok0 = b * _S + s0 + c * _CHUNK
        pltpu.sync_copy(x_hbm.at[pl.ds(tok0, _CHUNK)], ids_v)

        tb_vec = ids_v[pl.ds(0, 16)] * _D
        for t in range(_CHUNK):
            tb = tb_vec[t]
            ob = t * _D
            pb = c * _CHUNK * _D + ob

            @plsc.parallel_loop(0, _VPT, unroll=8)
            def _slice_body(j):
                o = j * 16
                buf[pl.ds(ob + o, 16)] = (
                    table_v[pl.ds(tb + o, 16)] + pe_v[pl.ds(pb + o, 16)]
                )

        pltpu.sync_copy(buf, out_hbm.at[pl.ds(tok0 * _D, _CHUNK * _D)])
        return carry

    lax.fori_loop(0, _B * n_chunks, chunk_body, 0)


def kernel(x, embedding_table):
    pe = _positional_encoding()
    out = _emb_kernel(
        x.reshape(-1), embedding_table.reshape(-1), pe.reshape(-1)
    )
    return out.reshape(_B, _S, _D)


# speedup vs baseline: 1.1044x; 1.1044x over previous
import functools

import jax
import jax.numpy as jnp
from jax import lax
from jax.experimental import pallas as pl
from jax.experimental.pallas import tpu as pltpu
from jax.experimental.pallas import tpu_sc as plsc

_VOCAB = 68
_D = 512
_S = 2048
_B = 64
_NW = 32
_S_PER_W = _S // _NW
_CHUNK = 32
_VPT = _D // 16
_CPB = _S_PER_W // _CHUNK


def _positional_encoding():
    pos = jnp.arange(0, _S, 1, dtype=jnp.float32).reshape(_S, 1)
    two_i = jnp.arange(0, _D, 2, dtype=jnp.float32)
    denominator = jnp.power(10000.0, two_i / _D)
    even = jnp.sin(pos / denominator)
    odd = jnp.cos(pos / denominator)
    return jnp.stack((even, odd), axis=2).reshape(_S, _D)


@functools.partial(
    pl.kernel,
    out_type=jax.ShapeDtypeStruct((_B * _S * _D,), jnp.float32),
    mesh=plsc.VectorSubcoreMesh(core_axis_name="c", subcore_axis_name="s"),
    scratch_types=[
        pltpu.VMEM((_VOCAB * _D,), jnp.float32),
        pltpu.VMEM((_S_PER_W * _D,), jnp.float32),
        pltpu.VMEM((_CHUNK * _D,), jnp.float32),
        pltpu.VMEM((_CHUNK * _D,), jnp.float32),
        pltpu.VMEM((_CHUNK,), jnp.int32),
        pltpu.VMEM((_CHUNK,), jnp.int32),
        pltpu.SemaphoreType.DMA,
        pltpu.SemaphoreType.DMA,
        pltpu.SemaphoreType.DMA,
        pltpu.SemaphoreType.DMA,
    ],
)
def _emb_kernel(
    x_hbm, table_hbm, pe_hbm, out_hbm,
    table_v, pe_v, buf0, buf1, ids0, ids1,
    sem_out0, sem_out1, sem_ids0, sem_ids1,
):
    wid = lax.axis_index("s") * 2 + lax.axis_index("c")
    s0 = wid * _S_PER_W

    pltpu.sync_copy(table_hbm, table_v)
    pltpu.sync_copy(pe_hbm.at[pl.ds(s0 * _D, _S_PER_W * _D)], pe_v)

    def tok0_of(i):
        return (i // _CPB) * _S + s0 + (i % _CPB) * _CHUNK

    def ids_start(i, ids_ref, sem):
        pltpu.make_async_copy(
            x_hbm.at[pl.ds(tok0_of(i), _CHUNK)], ids_ref, sem
        ).start()

    def ids_wait(ids_ref, sem):
        pltpu.make_async_copy(
            x_hbm.at[pl.ds(0, _CHUNK)], ids_ref, sem
        ).wait()

    def out_start(i, buf, sem):
        pltpu.make_async_copy(
            buf, out_hbm.at[pl.ds(tok0_of(i) * _D, _CHUNK * _D)], sem
        ).start()

    def out_wait(buf, sem):
        pltpu.make_async_copy(
            buf, out_hbm.at[pl.ds(0, _CHUNK * _D)], sem
        ).wait()

    def compute(i, buf, ids_ref):
        pb0 = (i % _CPB) * _CHUNK * _D
        for grp in range(_CHUNK // 16):
            tb16 = ids_ref[pl.ds(grp * 16, 16)] * _D
            tbs = [tb16[k] for k in range(16)]
            for k in range(16):
                ob = (grp * 16 + k) * _D
                pb = pb0 + ob

                @plsc.parallel_loop(0, _VPT, unroll=8)
                def _slice_body(j, tb=tbs[k], ob=ob, pb=pb):
                    o = j * 16
                    buf[pl.ds(ob + o, 16)] = (
                        table_v[pl.ds(tb + o, 16)] + pe_v[pl.ds(pb + o, 16)]
                    )

    n_pairs = _B * _CPB // 2
    ids_start(0, ids0, sem_ids0)

    def pair_body(g, carry):
        i0 = 2 * g
        i1 = i0 + 1
        ids_start(i1, ids1, sem_ids1)
        ids_wait(ids0, sem_ids0)

        @pl.when(g > 0)
        def _():
            out_wait(buf0, sem_out0)

        compute(i0, buf0, ids0)
        out_start(i0, buf0, sem_out0)

        @pl.when(g < n_pairs - 1)
        def _():
            ids_start(i0 + 2, ids0, sem_ids0)

        ids_wait(ids1, sem_ids1)

        @pl.when(g > 0)
        def _():
            out_wait(buf1, sem_out1)

        compute(i1, buf1, ids1)
        out_start(i1, buf1, sem_out1)
        return carry

    lax.fori_loop(0, n_pairs, pair_body, 0)
    out_wait(buf0, sem_out0)
    out_wait(buf1, sem_out1)


def kernel(x, embedding_table):
    pe = _positional_encoding()
    out = _emb_kernel(
        x.reshape(-1), embedding_table.reshape(-1), pe.reshape(-1)
    )
    return out.reshape(_B, _S, _D)


# speedup vs baseline: 1.8766x; 1.6992x over previous
import functools

import jax
import jax.numpy as jnp
from jax import lax
from jax.experimental import pallas as pl
from jax.experimental.pallas import tpu as pltpu
from jax.experimental.pallas import tpu_sc as plsc

_VOCAB = 68
_D = 512
_S = 2048
_B = 64
_NW = 32
_S_PER_W = _S // _NW
_CHUNK = 16
_VPT = _D // 16
_CPB = _S_PER_W // _CHUNK


def _positional_encoding():
    pos = jnp.arange(0, _S, 1, dtype=jnp.float32).reshape(_S, 1)
    two_i = jnp.arange(0, _D, 2, dtype=jnp.float32)
    denominator = jnp.power(10000.0, two_i / _D)
    even = jnp.sin(pos / denominator)
    odd = jnp.cos(pos / denominator)
    return jnp.stack((even, odd), axis=2).reshape(_S, _D)


@functools.partial(
    pl.kernel,
    out_type=jax.ShapeDtypeStruct((_B, _S, _D), jnp.float32),
    mesh=plsc.VectorSubcoreMesh(core_axis_name="c", subcore_axis_name="s"),
    compiler_params=pltpu.CompilerParams(use_tc_tiling_on_sc=True),
    scratch_types=[
        pltpu.VMEM((_VOCAB * _D,), jnp.float32),
        pltpu.VMEM((_S_PER_W * _D,), jnp.float32),
        pltpu.VMEM((_CHUNK, _D), jnp.float32),
        pltpu.VMEM((_CHUNK, _D), jnp.float32),
        pltpu.VMEM((_CHUNK,), jnp.int32),
        pltpu.VMEM((_CHUNK,), jnp.int32),
        pltpu.SemaphoreType.DMA,
        pltpu.SemaphoreType.DMA,
        pltpu.SemaphoreType.DMA,
        pltpu.SemaphoreType.DMA,
    ],
)
def _emb_kernel(
    x_hbm, table_hbm, pe_hbm, out_hbm,
    table_v, pe_v, buf0, buf1, ids0, ids1,
    sem_out0, sem_out1, sem_ids0, sem_ids1,
):
    wid = lax.axis_index("s") * 2 + lax.axis_index("c")
    s0 = wid * _S_PER_W

    pltpu.sync_copy(table_hbm, table_v)
    pltpu.sync_copy(pe_hbm.at[pl.ds(s0 * _D, _S_PER_W * _D)], pe_v)

    def ids_start(i, ids_ref, sem):
        tok0 = (i // _CPB) * _S + s0 + (i % _CPB) * _CHUNK
        pltpu.make_async_copy(
            x_hbm.at[pl.ds(tok0, _CHUNK)], ids_ref, sem
        ).start()

    def ids_wait(ids_ref, sem):
        pltpu.make_async_copy(
            x_hbm.at[pl.ds(0, _CHUNK)], ids_ref, sem
        ).wait()

    def out_start(i, buf, sem):
        b = i // _CPB
        s_abs = s0 + (i % _CPB) * _CHUNK
        pltpu.make_async_copy(
            buf, out_hbm.at[b, pl.ds(s_abs, _CHUNK), :], sem
        ).start()

    def out_wait(buf, sem):
        pltpu.make_async_copy(
            buf, out_hbm.at[0, pl.ds(0, _CHUNK), :], sem
        ).wait()

    def compute(i, buf, ids_ref):
        p0 = (i % _CPB) * _CHUNK
        tb16 = ids_ref[pl.ds(0, 16)] * _D
        tbs = [tb16[k] for k in range(16)]
        for k in range(_CHUNK):
            pb = (p0 + k) * _D

            @plsc.parallel_loop(0, _VPT, unroll=8)
            def _slice_body(j, tb=tbs[k], k=k, pb=pb):
                o = j * 16
                buf[k, pl.ds(o, 16)] = (
                    table_v[pl.ds(tb + o, 16)] + pe_v[pl.ds(pb + o, 16)]
                )

    n_pairs = _B * _CPB // 2
    ids_start(0, ids0, sem_ids0)

    def pair_body(g, carry):
        i0 = 2 * g
        i1 = i0 + 1
        ids_start(i1, ids1, sem_ids1)
        ids_wait(ids0, sem_ids0)

        @pl.when(g > 0)
        def _():
            out_wait(buf0, sem_out0)

        compute(i0, buf0, ids0)
        out_start(i0, buf0, sem_out0)

        @pl.when(g < n_pairs - 1)
        def _():
            ids_start(i0 + 2, ids0, sem_ids0)

        ids_wait(ids1, sem_ids1)

        @pl.when(g > 0)
        def _():
            out_wait(buf1, sem_out1)

        compute(i1, buf1, ids1)
        out_start(i1, buf1, sem_out1)
        return carry

    lax.fori_loop(0, n_pairs, pair_body, 0)
    out_wait(buf0, sem_out0)
    out_wait(buf1, sem_out1)


def kernel(x, embedding_table):
    pe = _positional_encoding()
    return _emb_kernel(
        x.reshape(-1), embedding_table.reshape(-1), pe.reshape(-1)
    )


# speedup vs baseline: 2.7959x; 1.4899x over previous
import functools

import jax
import jax.numpy as jnp
from jax import lax
from jax.experimental import pallas as pl
from jax.experimental.pallas import tpu as pltpu
from jax.experimental.pallas import tpu_sc as plsc

_VOCAB = 68
_D = 512
_S = 2048
_B = 64
_NW = 32
_S_PER_W = _S // _NW
_CHUNK = 16
_VPT = _D // 16
_CPB = _S_PER_W // _CHUNK


def _positional_encoding():
    pos = jnp.arange(0, _S, 1, dtype=jnp.float32).reshape(_S, 1)
    two_i = jnp.arange(0, _D, 2, dtype=jnp.float32)
    denominator = jnp.power(10000.0, two_i / _D)
    even = jnp.sin(pos / denominator)
    odd = jnp.cos(pos / denominator)
    return jnp.stack((even, odd), axis=2).reshape(_S, _D)


@functools.partial(
    pl.kernel,
    out_type=jax.ShapeDtypeStruct((_B, _S, _D), jnp.float32),
    mesh=plsc.VectorSubcoreMesh(core_axis_name="c", subcore_axis_name="s"),
    compiler_params=pltpu.CompilerParams(use_tc_tiling_on_sc=True),
    scratch_types=[
        pltpu.VMEM((_VOCAB * _D,), jnp.float32),
        pltpu.VMEM((_S_PER_W, _D), jnp.float32),
        pltpu.VMEM((_CHUNK, _D), jnp.float32),
        pltpu.VMEM((_CHUNK, _D), jnp.float32),
        pltpu.VMEM((_S_PER_W,), jnp.int32),
        pltpu.VMEM((_S_PER_W,), jnp.int32),
        pltpu.SemaphoreType.DMA,
        pltpu.SemaphoreType.DMA,
        pltpu.SemaphoreType.DMA,
        pltpu.SemaphoreType.DMA,
    ],
)
def _emb_kernel(
    x_hbm, table_hbm, pe_hbm, out_hbm,
    table_v, pe_v, buf0, buf1, ids0, ids1,
    sem_out0, sem_out1, sem_ids0, sem_ids1,
):
    wid = lax.axis_index("s") * 2 + lax.axis_index("c")
    s0 = wid * _S_PER_W

    pltpu.sync_copy(table_hbm, table_v)
    pltpu.sync_copy(pe_hbm.at[pl.ds(s0, _S_PER_W), :], pe_v)

    def ids_start(b, ids_ref, sem):
        pltpu.make_async_copy(
            x_hbm.at[b, pl.ds(s0, _S_PER_W)], ids_ref, sem
        ).start()

    def ids_wait(ids_ref, sem):
        pltpu.make_async_copy(
            x_hbm.at[0, pl.ds(0, _S_PER_W)], ids_ref, sem
        ).wait()

    def out_start(b, c, buf, sem):
        pltpu.make_async_copy(
            buf, out_hbm.at[b, pl.ds(s0 + c * _CHUNK, _CHUNK), :], sem
        ).start()

    def out_wait(buf, sem):
        pltpu.make_async_copy(
            buf, out_hbm.at[0, pl.ds(0, _CHUNK), :], sem
        ).wait()

    bufs = (buf0, buf1)
    out_sems = (sem_out0, sem_out1)

    def do_row(b, ids_ref, first):
        for c in range(_CPB):
            buf = bufs[c % 2]
            sem = out_sems[c % 2]
            if first and c < 2:
                @pl.when(b > 0)
                def _():
                    out_wait(buf, sem)
            else:
                out_wait(buf, sem)

            tb16 = ids_ref[pl.ds(c * 16, 16)] * _D
            tbs = [tb16[k] for k in range(16)]
            for k in range(_CHUNK):
                row = c * _CHUNK + k

                @plsc.parallel_loop(0, _VPT, unroll=8)
                def _slice_body(j, tb=tbs[k], k=k, row=row):
                    o = j * 16
                    buf[k, pl.ds(o, 16)] = (
                        table_v[pl.ds(tb + o, 16)] + pe_v[row, pl.ds(o, 16)]
                    )

            out_start(b, c, buf, sem)

    ids_start(0, ids0, sem_ids0)

    def pair_body(g, carry):
        b0 = 2 * g
        b1 = b0 + 1
        ids_start(b1, ids1, sem_ids1)
        ids_wait(ids0, sem_ids0)
        do_row(b0, ids0, first=True)

        @pl.when(g < _B // 2 - 1)
        def _():
            ids_start(b0 + 2, ids0, sem_ids0)

        ids_wait(ids1, sem_ids1)
        do_row(b1, ids1, first=False)
        return carry

    lax.fori_loop(0, _B // 2, pair_body, 0)
    out_wait(buf0, sem_out0)
    out_wait(buf1, sem_out1)


def kernel(x, embedding_table):
    pe = _positional_encoding()
    return _emb_kernel(x, embedding_table.reshape(-1), pe)


# speedup vs baseline: 3.4308x; 1.2271x over previous
import functools

import jax
import jax.numpy as jnp
from jax import lax
from jax.experimental import pallas as pl
from jax.experimental.pallas import tpu as pltpu
from jax.experimental.pallas import tpu_sc as plsc

_VOCAB = 68
_D = 512
_S = 2048
_B = 64
_NW = 32
_S_PER_W = _S // _NW
_CHUNK = 16
_VPT = _D // 16
_CPB = _S_PER_W // _CHUNK


def _positional_encoding():
    pos = jnp.arange(0, _S, 1, dtype=jnp.float32).reshape(_S, 1)
    two_i = jnp.arange(0, _D, 2, dtype=jnp.float32)
    denominator = jnp.power(10000.0, two_i / _D)
    even = jnp.sin(pos / denominator)
    odd = jnp.cos(pos / denominator)
    return jnp.stack((even, odd), axis=2).reshape(_S, _D)


@functools.partial(
    pl.kernel,
    out_type=jax.ShapeDtypeStruct((_B, _S, _D), jnp.float32),
    mesh=plsc.VectorSubcoreMesh(core_axis_name="c", subcore_axis_name="s"),
    compiler_params=pltpu.CompilerParams(use_tc_tiling_on_sc=True),
    scratch_types=[
        pltpu.VMEM((_VOCAB * _D,), jnp.float32),
        pltpu.VMEM_SHARED((_S // 2, _D), jnp.float32),
        pltpu.VMEM((_CHUNK, _D), jnp.float32),
        pltpu.VMEM((_CHUNK, _D), jnp.float32),
        pltpu.VMEM((_CHUNK, _D), jnp.float32),
        pltpu.VMEM((_CHUNK, _D), jnp.float32),
        pltpu.VMEM((_S_PER_W,), jnp.int32),
        pltpu.VMEM((_S_PER_W,), jnp.int32),
        pltpu.SemaphoreType.DMA,
        pltpu.SemaphoreType.DMA,
        pltpu.SemaphoreType.DMA,
        pltpu.SemaphoreType.DMA,
        pltpu.SemaphoreType.DMA,
        pltpu.SemaphoreType.DMA,
        pltpu.SemaphoreType.DMA,
        pltpu.SemaphoreType.DMA,
        pltpu.SemaphoreType.DMA,
        pltpu.SemaphoreType.DMA,
    ],
)
def _emb_kernel(
    x_hbm, table_hbm, pe_hbm, out_hbm,
    table_v, pe_sh, buf0, buf1, buf2, buf3, ids0, ids1,
    so0, so1, so2, so3, sp0, sp1, sp2, sp3, si0, si1,
):
    cid = lax.axis_index("c")
    sid = lax.axis_index("s")
    wid = sid * 2 + cid
    s0 = wid * _S_PER_W

    @pl.when(sid == 0)
    def _():
        for i in range(16):
            pltpu.sync_copy(
                pe_hbm.at[pl.ds((i * 2) * _S_PER_W + cid * _S_PER_W, _S_PER_W), :],
                pe_sh.at[pl.ds(i * _S_PER_W, _S_PER_W), :],
            )

    pltpu.sync_copy(table_hbm, table_v)
    plsc.subcore_barrier()

    bufs = (buf0, buf1, buf2, buf3)
    out_sems = (so0, so1, so2, so3)
    pre_sems = (sp0, sp1, sp2, sp3)

    def ids_start(b, ids_ref, sem):
        pltpu.make_async_copy(
            x_hbm.at[b, pl.ds(s0, _S_PER_W)], ids_ref, sem
        ).start()

    def ids_wait(ids_ref, sem):
        pltpu.make_async_copy(
            x_hbm.at[0, pl.ds(0, _S_PER_W)], ids_ref, sem
        ).wait()

    def pre_start(c, n):
        pltpu.make_async_copy(
            pe_sh.at[pl.ds(sid * _S_PER_W + c * _CHUNK, _CHUNK), :],
            bufs[n], pre_sems[n],
        ).start()

    def pre_wait(n):
        pltpu.make_async_copy(
            pe_sh.at[pl.ds(0, _CHUNK), :], bufs[n], pre_sems[n]
        ).wait()

    def out_start(b, c, n):
        pltpu.make_async_copy(
            bufs[n], out_hbm.at[b, pl.ds(s0 + c * _CHUNK, _CHUNK), :],
            out_sems[n],
        ).start()

    def out_wait(n):
        pltpu.make_async_copy(
            bufs[n], out_hbm.at[0, pl.ds(0, _CHUNK), :], out_sems[n]
        ).wait()

    def do_row(b, ids_ref):
        for c in range(_CPB):
            buf = bufs[c]
            pre_wait(c)

            tb16 = ids_ref[pl.ds(c * 16, 16)] * _D
            tbs = [tb16[k] for k in range(16)]
            for k0 in range(0, _CHUNK, 4):

                @plsc.parallel_loop(0, _VPT, unroll=4)
                def _slice_body(j, buf=buf, k0=k0):
                    o = j * 16
                    for k in range(k0, k0 + 4):
                        plsc.addupdate(
                            buf.at[k, pl.ds(o, 16)],
                            table_v[pl.ds(tbs[k] + o, 16)],
                        )

            out_start(b, c, c)

            n2 = (c + 2) % 4
            c2 = (c + 2) % _CPB
            if c < 2:
                @pl.when(b > 0)
                def _():
                    out_wait(n2)

                pre_start(c2, n2)
            else:
                @pl.when(b < _B - 1)
                def _():
                    out_wait(n2)
                    pre_start(c2, n2)

    ids_start(0, ids0, si0)
    pre_start(0, 0)
    pre_start(1, 1)

    def pair_body(g, carry):
        b0 = 2 * g
        b1 = b0 + 1
        ids_start(b1, ids1, si1)
        ids_wait(ids0, si0)
        do_row(b0, ids0)

        @pl.when(g < _B // 2 - 1)
        def _():
            ids_start(b0 + 2, ids0, si0)

        ids_wait(ids1, si1)
        do_row(b1, ids1)
        return carry

    lax.fori_loop(0, _B // 2, pair_body, 0)
    out_wait(0)
    out_wait(1)
    out_wait(2)
    out_wait(3)


def kernel(x, embedding_table):
    pe = _positional_encoding()
    return _emb_kernel(x, embedding_table.reshape(-1), pe)


# speedup vs baseline: 3.9584x; 1.1538x over previous
import functools

import jax
import jax.numpy as jnp
from jax import lax
from jax.experimental import pallas as pl
from jax.experimental.pallas import tpu as pltpu
from jax.experimental.pallas import tpu_sc as plsc

_VOCAB = 68
_D = 512
_S = 2048
_B = 64
_NW = 32
_S_PER_W = _S // _NW
_CHUNK = 16
_VPT = _D // 16
_CPB = _S_PER_W // _CHUNK


def _positional_encoding():
    pos = jnp.arange(0, _S, 1, dtype=jnp.float32).reshape(_S, 1)
    two_i = jnp.arange(0, _D, 2, dtype=jnp.float32)
    denominator = jnp.power(10000.0, two_i / _D)
    even = jnp.sin(pos / denominator)
    odd = jnp.cos(pos / denominator)
    return jnp.stack((even, odd), axis=2).reshape(_S, _D)


@functools.partial(
    pl.kernel,
    out_type=jax.ShapeDtypeStruct((_B, _S, _D), jnp.float32),
    mesh=plsc.VectorSubcoreMesh(core_axis_name="c", subcore_axis_name="s"),
    compiler_params=pltpu.CompilerParams(use_tc_tiling_on_sc=True),
    scratch_types=[
        pltpu.VMEM((_VOCAB * _D,), jnp.float32),
        pltpu.VMEM_SHARED((_S // 2, _D), jnp.float32),
        pltpu.VMEM((_CHUNK, _D), jnp.float32),
        pltpu.VMEM((_CHUNK, _D), jnp.float32),
        pltpu.VMEM((_CHUNK, _D), jnp.float32),
        pltpu.VMEM((_CHUNK, _D), jnp.float32),
        pltpu.VMEM((_S_PER_W,), jnp.int32),
        pltpu.VMEM((_S_PER_W,), jnp.int32),
        pltpu.SemaphoreType.DMA,
        pltpu.SemaphoreType.DMA,
        pltpu.SemaphoreType.DMA,
        pltpu.SemaphoreType.DMA,
        pltpu.SemaphoreType.DMA,
        pltpu.SemaphoreType.DMA,
        pltpu.SemaphoreType.DMA,
        pltpu.SemaphoreType.DMA,
        pltpu.SemaphoreType.DMA,
        pltpu.SemaphoreType.DMA,
    ],
)
def _emb_kernel(
    x_hbm, table_hbm, pe_hbm, out_hbm,
    table_v, pe_sh, buf0, buf1, buf2, buf3, ids0, ids1,
    so0, so1, so2, so3, sp0, sp1, sp2, sp3, si0, si1,
):
    cid = lax.axis_index("c")
    sid = lax.axis_index("s")
    wid = sid * 2 + cid
    s0 = wid * _S_PER_W

    @pl.when(sid == 0)
    def _():
        for i in range(16):
            pltpu.sync_copy(
                pe_hbm.at[pl.ds((i * 2) * _S_PER_W + cid * _S_PER_W, _S_PER_W), :],
                pe_sh.at[pl.ds(i * _S_PER_W, _S_PER_W), :],
            )

    pltpu.sync_copy(table_hbm, table_v)
    plsc.subcore_barrier()

    bufs = (buf0, buf1, buf2, buf3)
    out_sems = (so0, so1, so2, so3)
    pre_sems = (sp0, sp1, sp2, sp3)

    def ids_start(b, ids_ref, sem):
        pltpu.make_async_copy(
            x_hbm.at[b, pl.ds(s0, _S_PER_W)], ids_ref, sem
        ).start()

    def ids_wait(ids_ref, sem):
        pltpu.make_async_copy(
            x_hbm.at[0, pl.ds(0, _S_PER_W)], ids_ref, sem
        ).wait()

    def pre_start(c, n):
        pltpu.make_async_copy(
            pe_sh.at[pl.ds(sid * _S_PER_W + c * _CHUNK, _CHUNK), :],
            bufs[n], pre_sems[n],
        ).start()

    def pre_wait(n):
        pltpu.make_async_copy(
            pe_sh.at[pl.ds(0, _CHUNK), :], bufs[n], pre_sems[n]
        ).wait()

    def out_start(b, c, n):
        pltpu.make_async_copy(
            bufs[n], out_hbm.at[b, pl.ds(s0 + c * _CHUNK, _CHUNK), :],
            out_sems[n],
        ).start()

    def out_wait(n):
        pltpu.make_async_copy(
            bufs[n], out_hbm.at[0, pl.ds(0, _CHUNK), :], out_sems[n]
        ).wait()

    def do_row(b, ids_ref):
        for c in range(_CPB):
            buf = bufs[c]
            pre_wait(c)

            tb16 = ids_ref[pl.ds(c * 16, 16)] * _D
            tbs = [tb16[k] for k in range(16)]
            for k0 in range(0, _CHUNK, 8):

                @plsc.parallel_loop(0, _VPT, unroll=2)
                def _slice_body(j, buf=buf, k0=k0):
                    o = j * 16
                    for k in range(k0, k0 + 8):
                        plsc.addupdate(
                            buf.at[k, pl.ds(o, 16)],
                            table_v[pl.ds(tbs[k] + o, 16)],
                        )

            out_start(b, c, c)

            n2 = (c + 2) % 4
            c2 = (c + 2) % _CPB
            if c < 2:
                @pl.when(b > 0)
                def _():
                    out_wait(n2)

                pre_start(c2, n2)
            else:
                @pl.when(b < _B - 1)
                def _():
                    out_wait(n2)
                    pre_start(c2, n2)

    ids_start(0, ids0, si0)
    pre_start(0, 0)
    pre_start(1, 1)

    def pair_body(g, carry):
        b0 = 2 * g
        b1 = b0 + 1
        ids_start(b1, ids1, si1)
        ids_wait(ids0, si0)
        do_row(b0, ids0)

        @pl.when(g < _B // 2 - 1)
        def _():
            ids_start(b0 + 2, ids0, si0)

        ids_wait(ids1, si1)
        do_row(b1, ids1)
        return carry

    lax.fori_loop(0, _B // 2, pair_body, 0)
    out_wait(0)
    out_wait(1)
    out_wait(2)
    out_wait(3)


def kernel(x, embedding_table):
    pe = _positional_encoding()
    return _emb_kernel(x, embedding_table.reshape(-1), pe)


# speedup vs baseline: 4.0570x; 1.0249x over previous
import functools

import jax
import jax.numpy as jnp
from jax import lax
from jax.experimental import pallas as pl
from jax.experimental.pallas import tpu as pltpu
from jax.experimental.pallas import tpu_sc as plsc

_VOCAB = 68
_D = 512
_S = 2048
_B = 64
_NW = 32
_S_PER_W = _S // _NW
_CHUNK = 16
_VPT = _D // 16
_CPB = _S_PER_W // _CHUNK


def _positional_encoding():
    pos = jnp.arange(0, _S, 1, dtype=jnp.float32).reshape(_S, 1)
    two_i = jnp.arange(0, _D, 2, dtype=jnp.float32)
    denominator = jnp.power(10000.0, two_i / _D)
    even = jnp.sin(pos / denominator)
    odd = jnp.cos(pos / denominator)
    return jnp.stack((even, odd), axis=2).reshape(_S, _D)


@functools.partial(
    pl.kernel,
    out_type=jax.ShapeDtypeStruct((_B, _S, _D), jnp.float32),
    mesh=plsc.VectorSubcoreMesh(core_axis_name="c", subcore_axis_name="s"),
    compiler_params=pltpu.CompilerParams(use_tc_tiling_on_sc=True),
    scratch_types=[
        pltpu.VMEM((_VOCAB * _D,), jnp.float32),
        pltpu.VMEM_SHARED((_S // 2, _D), jnp.float32),
        pltpu.VMEM((_CHUNK, _D), jnp.float32),
        pltpu.VMEM((_CHUNK, _D), jnp.float32),
        pltpu.VMEM((_CHUNK, _D), jnp.float32),
        pltpu.VMEM((_CHUNK, _D), jnp.float32),
        pltpu.VMEM((_S_PER_W,), jnp.int32),
        pltpu.VMEM((_S_PER_W,), jnp.int32),
        pltpu.SemaphoreType.DMA,
        pltpu.SemaphoreType.DMA,
        pltpu.SemaphoreType.DMA,
        pltpu.SemaphoreType.DMA,
        pltpu.SemaphoreType.DMA,
        pltpu.SemaphoreType.DMA,
        pltpu.SemaphoreType.DMA,
        pltpu.SemaphoreType.DMA,
        pltpu.SemaphoreType.DMA,
        pltpu.SemaphoreType.DMA,
    ],
)
def _emb_kernel(
    x_hbm, table_hbm, pe_hbm, out_hbm,
    table_v, pe_sh, buf0, buf1, buf2, buf3, ids0, ids1,
    so0, so1, so2, so3, sp0, sp1, sp2, sp3, si0, si1,
):
    cid = lax.axis_index("c")
    sid = lax.axis_index("s")
    wid = sid * 2 + cid
    s0 = wid * _S_PER_W

    @pl.when(sid == 0)
    def _():
        for i in range(16):
            pltpu.sync_copy(
                pe_hbm.at[pl.ds((i * 2) * _S_PER_W + cid * _S_PER_W, _S_PER_W), :],
                pe_sh.at[pl.ds(i * _S_PER_W, _S_PER_W), :],
            )

    pltpu.sync_copy(table_hbm, table_v)
    plsc.subcore_barrier()

    bufs = (buf0, buf1, buf2, buf3)
    out_sems = (so0, so1, so2, so3)
    pre_sems = (sp0, sp1, sp2, sp3)

    def ids_start(b, ids_ref, sem):
        pltpu.make_async_copy(
            x_hbm.at[b, pl.ds(s0, _S_PER_W)], ids_ref, sem
        ).start()

    def ids_wait(ids_ref, sem):
        pltpu.make_async_copy(
            x_hbm.at[0, pl.ds(0, _S_PER_W)], ids_ref, sem
        ).wait()

    def pre_start(c, n):
        pltpu.make_async_copy(
            pe_sh.at[pl.ds(sid * _S_PER_W + c * _CHUNK, _CHUNK), :],
            bufs[n], pre_sems[n],
        ).start()

    def pre_wait(n):
        pltpu.make_async_copy(
            pe_sh.at[pl.ds(0, _CHUNK), :], bufs[n], pre_sems[n]
        ).wait()

    def out_start(b, c, n):
        pltpu.make_async_copy(
            bufs[n], out_hbm.at[b, pl.ds(s0 + c * _CHUNK, _CHUNK), :],
            out_sems[n],
        ).start()

    def out_wait(n):
        pltpu.make_async_copy(
            bufs[n], out_hbm.at[0, pl.ds(0, _CHUNK), :], out_sems[n]
        ).wait()

    def do_row(b, ids_ref):
        for c in range(_CPB):
            buf = bufs[c]
            pre_wait(c)

            tb16 = ids_ref[pl.ds(c * 16, 16)] * _D
            tbs = [tb16[k] for k in range(16)]
            @plsc.parallel_loop(0, _VPT, unroll=1)
            def _slice_body(j, buf=buf):
                o = j * 16
                for k in range(_CHUNK):
                    plsc.addupdate(
                        buf.at[k, pl.ds(o, 16)],
                        table_v[pl.ds(tbs[k] + o, 16)],
                    )

            out_start(b, c, c)

            n2 = (c + 2) % 4
            c2 = (c + 2) % _CPB
            if c < 2:
                @pl.when(b > 0)
                def _():
                    out_wait(n2)

                pre_start(c2, n2)
            else:
                @pl.when(b < _B - 1)
                def _():
                    out_wait(n2)
                    pre_start(c2, n2)

    ids_start(0, ids0, si0)
    pre_start(0, 0)
    pre_start(1, 1)

    def pair_body(g, carry):
        b0 = 2 * g
        b1 = b0 + 1
        ids_start(b1, ids1, si1)
        ids_wait(ids0, si0)
        do_row(b0, ids0)

        @pl.when(g < _B // 2 - 1)
        def _():
            ids_start(b0 + 2, ids0, si0)

        ids_wait(ids1, si1)
        do_row(b1, ids1)
        return carry

    lax.fori_loop(0, _B // 2, pair_body, 0)
    out_wait(0)
    out_wait(1)
    out_wait(2)
    out_wait(3)


def kernel(x, embedding_table):
    pe = _positional_encoding()
    return _emb_kernel(x, embedding_table.reshape(-1), pe)
